# baseline (speedup 1.0000x reference)
import functools

import jax
import jax.numpy as jnp
from jax import lax
from jax.experimental import pallas as pl
from jax.experimental.pallas import tpu as pltpu
from jax.experimental.pallas import tpu_sc as plsc

N_NODES = 10000
N_EDGES = 320000
D = 128
N_GRAPHS = 64

_NC = 2
_NS = 16
_NW = _NC * _NS
_K = 125
_NCHUNK = N_EDGES // (_NW * _K)
_NHALF = _NCHUNK // 2
_RPT = (N_NODES // _NS) // 8 * 8
_REM = N_NODES - _NS * _RPT

_BLK = 1000
_NBLK = N_NODES // _BLK


def _sc_agg_body(h_hbm, src_hbm, dst_hbm, zero_hbm, out_hbm,
                 sidx_v, didx_v, rows0, rows1, gs0, gs1, ss0, ss1, agg_sh):
    c = lax.axis_index("c")
    s = lax.axis_index("s")
    wid = c * _NS + s
    r0 = s * _RPT

    @pl.when(c == 0)
    def _():
        pltpu.sync_copy(h_hbm.at[pl.ds(r0, _RPT)], agg_sh.at[pl.ds(r0, _RPT)])

        @pl.when(s == _NS - 1)
        def _():
            pltpu.sync_copy(h_hbm.at[pl.ds(_NS * _RPT, _REM)],
                            agg_sh.at[pl.ds(_NS * _RPT, _REM)])

    @pl.when(c != 0)
    def _():
        pltpu.sync_copy(zero_hbm.at[pl.ds(r0, _RPT)], agg_sh.at[pl.ds(r0, _RPT)])

        @pl.when(s == _NS - 1)
        def _():
            pltpu.sync_copy(zero_hbm.at[pl.ds(_NS * _RPT, _REM)],
                            agg_sh.at[pl.ds(_NS * _RPT, _REM)])

    pltpu.sync_copy(src_hbm.at[wid], sidx_v)

    plsc.subcore_barrier()

    def gather(j, buf, sem):
        return pltpu.make_async_copy(h_hbm.at[sidx_v.at[j]], buf, sem)

    def scatter_start(k, buf, sem):
        pltpu.async_copy(buf, agg_sh.at[didx_v.at[k]], sem, add=True)

    def scatter_wait(k, buf, sem):
        pltpu.make_async_copy(buf, agg_sh.at[didx_v.at[k]], sem).wait()

    for half in range(2):
        base = half * _NHALF
        pltpu.sync_copy(dst_hbm.at[wid, pl.ds(base, _NHALF)], didx_v)
        gather(base, rows0, gs0).start()
        gather(base + 1, rows1, gs1).start()

        def chunk(i, carry):
            j = base + 2 * i
            k = 2 * i
            gather(j, rows0, gs0).wait()
            scatter_start(k, rows0, ss0)
            gather(j + 1, rows1, gs1).wait()
            scatter_start(k + 1, rows1, ss1)

            @pl.when(k + 2 < _NHALF)
            def _():
                scatter_wait(k, rows0, ss0)
                gather(j + 2, rows0, gs0).start()
                scatter_wait(k + 1, rows1, ss1)
                gather(j + 3, rows1, gs1).start()

            return carry

        lax.fori_loop(0, _NHALF // 2, chunk, 0)
        scatter_wait(_NHALF - 2, rows0, ss0)
        scatter_wait(_NHALF - 1, rows1, ss1)

    plsc.subcore_barrier()
    pltpu.sync_copy(agg_sh.at[pl.ds(r0, _RPT)],
                    out_hbm.at[c, pl.ds(r0, _RPT)])

    @pl.when(s == _NS - 1)
    def _():
        pltpu.sync_copy(agg_sh.at[pl.ds(_NS * _RPT, _REM)],
                        out_hbm.at[c, pl.ds(_NS * _RPT, _REM)])


@functools.cache
def _sc_agg_kernel():
    return pl.kernel(
        _sc_agg_body,
        out_type=jax.ShapeDtypeStruct((_NC, N_NODES, D), jnp.float32),
        mesh=plsc.VectorSubcoreMesh(core_axis_name="c", subcore_axis_name="s",
                                    num_cores=_NC, num_subcores=_NS),
        scratch_types=[
            pltpu.VMEM((_NCHUNK, _K), jnp.int32),
            pltpu.VMEM((_NHALF, _K), jnp.int32),
            pltpu.VMEM((_K, D), jnp.float32),
            pltpu.VMEM((_K, D), jnp.float32),
            pltpu.SemaphoreType.DMA,
            pltpu.SemaphoreType.DMA,
            pltpu.SemaphoreType.DMA,
            pltpu.SemaphoreType.DMA,
            pltpu.VMEM_SHARED((N_NODES, D), jnp.float32),
        ],
    )


def _sc_agg(h, srcp, dstp, zeros):
    return _sc_agg_kernel()(h, srcp, dstp, zeros)


def _mm_body(p_ref, w_ref, b_ref, o_ref):
    hsum = p_ref[0] + p_ref[1]
    o_ref[...] = jnp.maximum(
        jax.lax.dot(hsum, w_ref[...], preferred_element_type=jnp.float32)
        + b_ref[...], 0.0)


def _mm(p, W, b2d):
    return pl.pallas_call(
        _mm_body,
        grid=(_NBLK,),
        in_specs=[
            pl.BlockSpec((_NC, _BLK, D), lambda i: (0, i, 0)),
            pl.BlockSpec((D, D), lambda i: (0, 0)),
            pl.BlockSpec((1, D), lambda i: (0, 0)),
        ],
        out_specs=pl.BlockSpec((_BLK, D), lambda i: (i, 0)),
        out_shape=jax.ShapeDtypeStruct((N_NODES, D), jnp.float32),
    )(p, W, b2d)


def _final_body(p_ref, w3_ref, b3_ref, batch_ref, wfc_ref, bfc_ref, o_ref,
                sums, counts):
    i = pl.program_id(0)

    @pl.when(i == 0)
    def _():
        sums[...] = jnp.zeros_like(sums)
        counts[...] = jnp.zeros_like(counts)

    h3 = jnp.maximum(
        jax.lax.dot(p_ref[0] + p_ref[1], w3_ref[...],
                    preferred_element_type=jnp.float32) + b3_ref[...], 0.0)
    bb = batch_ref[0]
    onehot = (lax.broadcasted_iota(jnp.int32, (N_GRAPHS, _BLK), 0)
              == bb).astype(jnp.float32)
    sums[...] += jax.lax.dot(onehot, h3, preferred_element_type=jnp.float32)
    counts[...] += jnp.sum(onehot, axis=1, keepdims=True)

    @pl.when(i == pl.num_programs(0) - 1)
    def _():
        pooled = sums[...] / jnp.maximum(counts[...], 1.0)
        o_ref[...] = (
            jax.lax.dot(pooled, wfc_ref[...],
                        preferred_element_type=jnp.float32) + bfc_ref[...])


def _final(p, W3, b32d, batch3, Wfc, bfc2d):
    return pl.pallas_call(
        _final_body,
        grid=(_NBLK,),
        in_specs=[
            pl.BlockSpec((_NC, _BLK, D), lambda i: (0, i, 0)),
            pl.BlockSpec((D, D), lambda i: (0, 0)),
            pl.BlockSpec((1, D), lambda i: (0, 0)),
            pl.BlockSpec((1, 1, _BLK), lambda i: (i, 0, 0)),
            pl.BlockSpec((D, D), lambda i: (0, 0)),
            pl.BlockSpec((1, D), lambda i: (0, 0)),
        ],
        out_specs=pl.BlockSpec((N_GRAPHS, D), lambda i: (0, 0)),
        out_shape=jax.ShapeDtypeStruct((N_GRAPHS, D), jnp.float32),
        scratch_shapes=[
            pltpu.VMEM((N_GRAPHS, D), jnp.float32),
            pltpu.VMEM((N_GRAPHS, 1), jnp.float32),
        ],
    )(p, W3, b32d, batch3, Wfc, bfc2d)


def kernel(x, edge_index, batch, W1, b1, W2, b2, W3, b3, Wfc, bfc):
    edge_index = edge_index.astype(jnp.int32)
    srcp = edge_index[0].reshape(_NW, _NCHUNK, _K)
    dstp = edge_index[1].reshape(_NW, _NCHUNK, _K)
    zeros = jnp.zeros((N_NODES, D), jnp.float32)
    batch3 = batch.astype(jnp.int32).reshape(_NBLK, 1, _BLK)
    b1r = b1.reshape(1, D)
    b2r = b2.reshape(1, D)
    b3r = b3.reshape(1, D)
    bfcr = bfc.reshape(1, D)

    p = _sc_agg(x, srcp, dstp, zeros)
    h = _mm(p, W1, b1r)
    p = _sc_agg(h, srcp, dstp, zeros)
    h = _mm(p, W2, b2r)
    p = _sc_agg(h, srcp, dstp, zeros)
    return _final(p, W3, b3r, batch3, Wfc, bfcr)


# speedup vs baseline: 1.1162x; 1.1162x over previous
import functools

import jax
import jax.numpy as jnp
from jax import lax
from jax.experimental import pallas as pl
from jax.experimental.pallas import tpu as pltpu
from jax.experimental.pallas import tpu_sc as plsc

N_NODES = 10000
N_EDGES = 320000
D = 128
N_GRAPHS = 64

_NC = 2
_NS = 16
_NW = _NC * _NS
_K = 125
_NCHUNK = N_EDGES // (_NW * _K)
_NHALF = _NCHUNK // 2
_RPT = (N_NODES // _NS) // 8 * 8
_REM = N_NODES - _NS * _RPT

_BLK = 2000
_NBLK = N_NODES // _BLK


def _sc_agg_body(h_hbm, src_hbm, dst_hbm, zero_hbm, out_hbm,
                 sidx_v, didx_v, rows0, rows1, gs0, gs1, agg_sh):
    c = lax.axis_index("c")
    s = lax.axis_index("s")
    wid = c * _NS + s
    r0 = s * _RPT

    @pl.when(c == 0)
    def _():
        pltpu.make_async_copy(h_hbm.at[pl.ds(r0, _RPT)],
                              agg_sh.at[pl.ds(r0, _RPT)], gs1).start()

        @pl.when(s == _NS - 1)
        def _():
            pltpu.sync_copy(h_hbm.at[pl.ds(_NS * _RPT, _REM)],
                            agg_sh.at[pl.ds(_NS * _RPT, _REM)])

    @pl.when(c != 0)
    def _():
        pltpu.make_async_copy(zero_hbm.at[pl.ds(r0, _RPT)],
                              agg_sh.at[pl.ds(r0, _RPT)], gs1).start()

        @pl.when(s == _NS - 1)
        def _():
            pltpu.sync_copy(zero_hbm.at[pl.ds(_NS * _RPT, _REM)],
                            agg_sh.at[pl.ds(_NS * _RPT, _REM)])

    pltpu.sync_copy(src_hbm.at[wid], sidx_v)

    @pl.when(c == 0)
    def _():
        pltpu.make_async_copy(h_hbm.at[pl.ds(r0, _RPT)],
                              agg_sh.at[pl.ds(r0, _RPT)], gs1).wait()

    @pl.when(c != 0)
    def _():
        pltpu.make_async_copy(zero_hbm.at[pl.ds(r0, _RPT)],
                              agg_sh.at[pl.ds(r0, _RPT)], gs1).wait()

    plsc.subcore_barrier()

    def gather(j, buf, sem):
        return pltpu.make_async_copy(h_hbm.at[sidx_v.at[j]], buf, sem)

    for half in range(2):
        base = half * _NHALF
        pltpu.sync_copy(dst_hbm.at[wid, pl.ds(base, _NHALF)], didx_v)
        gather(base, rows0, gs0).start()

        def chunk(i, carry):
            j = base + 2 * i
            gather(j, rows0, gs0).wait()
            gather(j + 1, rows1, gs1).start()
            pltpu.sync_copy(rows0, agg_sh.at[didx_v.at[2 * i]], add=True)
            gather(j + 1, rows1, gs1).wait()

            @pl.when(2 * i + 2 < _NHALF)
            def _():
                gather(j + 2, rows0, gs0).start()

            pltpu.sync_copy(rows1, agg_sh.at[didx_v.at[2 * i + 1]], add=True)
            return carry

        lax.fori_loop(0, _NHALF // 2, chunk, 0)

    plsc.subcore_barrier()
    pltpu.sync_copy(agg_sh.at[pl.ds(r0, _RPT)],
                    out_hbm.at[c, pl.ds(r0, _RPT)])

    @pl.when(s == _NS - 1)
    def _():
        pltpu.sync_copy(agg_sh.at[pl.ds(_NS * _RPT, _REM)],
                        out_hbm.at[c, pl.ds(_NS * _RPT, _REM)])


@functools.cache
def _sc_agg_kernel():
    return pl.kernel(
        _sc_agg_body,
        out_type=jax.ShapeDtypeStruct((_NC, N_NODES, D), jnp.float32),
        mesh=plsc.VectorSubcoreMesh(core_axis_name="c", subcore_axis_name="s",
                                    num_cores=_NC, num_subcores=_NS),
        scratch_types=[
            pltpu.VMEM((_NCHUNK, _K), jnp.int32),
            pltpu.VMEM((_NHALF, _K), jnp.int32),
            pltpu.VMEM((_K, D), jnp.float32),
            pltpu.VMEM((_K, D), jnp.float32),
            pltpu.SemaphoreType.DMA,
            pltpu.SemaphoreType.DMA,
            pltpu.VMEM_SHARED((N_NODES, D), jnp.float32),
        ],
    )


def _sc_agg(h, srcp, dstp, zeros):
    return _sc_agg_kernel()(h, srcp, dstp, zeros)


def _mm_body(p_ref, w_ref, b_ref, o_ref):
    hsum = p_ref[0] + p_ref[1]
    o_ref[...] = jnp.maximum(
        jax.lax.dot(hsum, w_ref[...], preferred_element_type=jnp.float32)
        + b_ref[...], 0.0)


def _mm(p, W, b2d):
    return pl.pallas_call(
        _mm_body,
        grid=(_NBLK,),
        in_specs=[
            pl.BlockSpec((_NC, _BLK, D), lambda i: (0, i, 0)),
            pl.BlockSpec((D, D), lambda i: (0, 0)),
            pl.BlockSpec((1, D), lambda i: (0, 0)),
        ],
        out_specs=pl.BlockSpec((_BLK, D), lambda i: (i, 0)),
        out_shape=jax.ShapeDtypeStruct((N_NODES, D), jnp.float32),
    )(p, W, b2d)


def _final_body(p_ref, w3_ref, b3_ref, batch_ref, wfc_ref, bfc_ref, o_ref,
                sums, counts):
    i = pl.program_id(0)

    @pl.when(i == 0)
    def _():
        sums[...] = jnp.zeros_like(sums)
        counts[...] = jnp.zeros_like(counts)

    h3 = jnp.maximum(
        jax.lax.dot(p_ref[0] + p_ref[1], w3_ref[...],
                    preferred_element_type=jnp.float32) + b3_ref[...], 0.0)
    bb = batch_ref[0]
    onehot = (lax.broadcasted_iota(jnp.int32, (N_GRAPHS, _BLK), 0)
              == bb).astype(jnp.float32)
    sums[...] += jax.lax.dot(onehot, h3, preferred_element_type=jnp.float32)
    counts[...] += jnp.sum(onehot, axis=1, keepdims=True)

    @pl.when(i == pl.num_programs(0) - 1)
    def _():
        pooled = sums[...] / jnp.maximum(counts[...], 1.0)
        o_ref[...] = (
            jax.lax.dot(pooled, wfc_ref[...],
                        preferred_element_type=jnp.float32) + bfc_ref[...])


def _final(p, W3, b32d, batch3, Wfc, bfc2d):
    return pl.pallas_call(
        _final_body,
        grid=(_NBLK,),
        in_specs=[
            pl.BlockSpec((_NC, _BLK, D), lambda i: (0, i, 0)),
            pl.BlockSpec((D, D), lambda i: (0, 0)),
            pl.BlockSpec((1, D), lambda i: (0, 0)),
            pl.BlockSpec((1, 1, _BLK), lambda i: (i, 0, 0)),
            pl.BlockSpec((D, D), lambda i: (0, 0)),
            pl.BlockSpec((1, D), lambda i: (0, 0)),
        ],
        out_specs=pl.BlockSpec((N_GRAPHS, D), lambda i: (0, 0)),
        out_shape=jax.ShapeDtypeStruct((N_GRAPHS, D), jnp.float32),
        scratch_shapes=[
            pltpu.VMEM((N_GRAPHS, D), jnp.float32),
            pltpu.VMEM((N_GRAPHS, 1), jnp.float32),
        ],
    )(p, W3, b32d, batch3, Wfc, bfc2d)


def kernel(x, edge_index, batch, W1, b1, W2, b2, W3, b3, Wfc, bfc):
    edge_index = edge_index.astype(jnp.int32)
    srcp = edge_index[0].reshape(_NW, _NCHUNK, _K)
    dstp = edge_index[1].reshape(_NW, _NCHUNK, _K)
    zeros = jnp.zeros((N_NODES, D), jnp.float32)
    batch3 = batch.astype(jnp.int32).reshape(_NBLK, 1, _BLK)
    b1r = b1.reshape(1, D)
    b2r = b2.reshape(1, D)
    b3r = b3.reshape(1, D)
    bfcr = bfc.reshape(1, D)

    p = _sc_agg(x, srcp, dstp, zeros)
    h = _mm(p, W1, b1r)
    p = _sc_agg(h, srcp, dstp, zeros)
    h = _mm(p, W2, b2r)
    p = _sc_agg(h, srcp, dstp, zeros)
    return _final(p, W3, b3r, batch3, Wfc, bfcr)
